# baseline (speedup 1.0000x reference)
import functools

import jax
import jax.numpy as jnp
from jax import lax
from jax.experimental import pallas as pl
from jax.experimental.pallas import tpu as pltpu
from jax.experimental.pallas import tpu_sc as plsc

_NC, _NS = 2, 16


def _make_sc_lookup(D, N):
    n_chunks = D // 8
    mesh = plsc.VectorSubcoreMesh(
        core_axis_name="c", subcore_axis_name="s", num_cores=_NC, num_subcores=_NS
    )

    @functools.partial(
        pl.kernel,
        mesh=mesh,
        out_type=jax.ShapeDtypeStruct((D, N), jnp.float32),
        scratch_types=[
            pltpu.VMEM((8, N), jnp.float32),
            pltpu.SemaphoreType.DMA,
        ],
        compiler_params=pltpu.CompilerParams(use_tc_tiling_on_sc=True),
    )
    def k(t_hbm, o_hbm, buf, sem):
        c = lax.axis_index("c")
        s = lax.axis_index("s")
        wid = s * _NC + c

        @pl.when(wid < n_chunks)
        def _lookup():
            r0 = pl.multiple_of(wid * 8, 8)
            pltpu.make_async_copy(t_hbm.at[pl.ds(r0, 8)], buf, sem).start()
            pltpu.make_async_copy(t_hbm.at[pl.ds(r0, 8)], buf, sem).wait()
            pltpu.make_async_copy(buf, o_hbm.at[pl.ds(r0, 8)], sem).start()
            pltpu.make_async_copy(buf, o_hbm.at[pl.ds(r0, 8)], sem).wait()

    return k


def _make_tc_body(B, T):
    def body(t_ref, o_ref, sem):
        def fire(i, c):
            b = i // T
            t = i - b * T
            pltpu.make_async_copy(t_ref, o_ref.at[b, t], sem).start()
            return c

        lax.fori_loop(0, B * T, fire, 0)

        def drain(i, c):
            b = i // T
            t = i - b * T
            pltpu.make_async_copy(t_ref, o_ref.at[b, t], sem).wait()
            return c

        lax.fori_loop(0, B * T, drain, 0)

    return body


def _merge_body(big_ref, slab_ref, o_ref, sem):
    del big_ref
    pltpu.make_async_copy(slab_ref, o_ref.at[0, 0], sem).start()
    pltpu.make_async_copy(slab_ref, o_ref.at[0, 0], sem).wait()


def kernel(x, table):
    B, T, N, _ = x.shape
    D = table.shape[1]
    t2 = table.T
    slab = _make_sc_lookup(D, N)(t2)
    out = pl.pallas_call(
        _make_tc_body(B, T),
        in_specs=[pl.BlockSpec(memory_space=pltpu.VMEM)],
        out_specs=pl.BlockSpec(memory_space=pl.ANY),
        out_shape=jax.ShapeDtypeStruct((B, T, D, N), jnp.float32),
        scratch_shapes=[pltpu.SemaphoreType.DMA],
    )(t2)
    out = pl.pallas_call(
        _merge_body,
        in_specs=[
            pl.BlockSpec(memory_space=pl.ANY),
            pl.BlockSpec(memory_space=pl.ANY),
        ],
        out_specs=pl.BlockSpec(memory_space=pl.ANY),
        out_shape=jax.ShapeDtypeStruct((B, T, D, N), jnp.float32),
        scratch_shapes=[pltpu.SemaphoreType.DMA],
        input_output_aliases={0: 0},
    )(out, slab)
    return out.transpose(0, 1, 3, 2)


# speedup vs baseline: 1.3859x; 1.3859x over previous
import functools

import jax
import jax.numpy as jnp
from jax import lax
from jax.experimental import pallas as pl
from jax.experimental.pallas import tpu as pltpu
from jax.experimental.pallas import tpu_sc as plsc

_NC, _NS = 2, 16


def _make_sc_lookup(D, N):
    n_chunks = D // 8
    mesh = plsc.VectorSubcoreMesh(
        core_axis_name="c", subcore_axis_name="s", num_cores=_NC, num_subcores=_NS
    )

    @functools.partial(
        pl.kernel,
        mesh=mesh,
        out_type=jax.ShapeDtypeStruct((D, N), jnp.float32),
        scratch_types=[
            pltpu.VMEM((8, N), jnp.float32),
            pltpu.SemaphoreType.DMA,
        ],
        compiler_params=pltpu.CompilerParams(use_tc_tiling_on_sc=True),
    )
    def k(t_hbm, o_hbm, buf, sem):
        c = lax.axis_index("c")
        s = lax.axis_index("s")
        wid = s * _NC + c

        @pl.when(wid < n_chunks)
        def _lookup():
            r0 = pl.multiple_of(wid * 8, 8)
            pltpu.make_async_copy(t_hbm.at[pl.ds(r0, 8)], buf, sem).start()
            pltpu.make_async_copy(t_hbm.at[pl.ds(r0, 8)], buf, sem).wait()
            pltpu.make_async_copy(buf, o_hbm.at[pl.ds(r0, 8)], sem).start()
            pltpu.make_async_copy(buf, o_hbm.at[pl.ds(r0, 8)], sem).wait()

    return k


def _make_tc_body(B, T):
    def body(t_ref, o_ref, pair, sem):
        pltpu.make_async_copy(t_ref, pair.at[0], sem).start()
        pltpu.make_async_copy(t_ref, pair.at[1], sem).start()
        pltpu.make_async_copy(t_ref, pair.at[0], sem).wait()
        pltpu.make_async_copy(t_ref, pair.at[1], sem).wait()
        n = B * T // 2

        def fire(i, c):
            b = i // (T // 2)
            t = (i - b * (T // 2)) * 2
            pltpu.make_async_copy(pair, o_ref.at[b, pl.ds(t, 2)], sem).start()
            return c

        lax.fori_loop(0, n, fire, 0)

        def drain(i, c):
            b = i // (T // 2)
            t = (i - b * (T // 2)) * 2
            pltpu.make_async_copy(pair, o_ref.at[b, pl.ds(t, 2)], sem).wait()
            return c

        lax.fori_loop(0, n, drain, 0)

    return body


def _merge_body(big_ref, slab_ref, o_ref, sem):
    del big_ref
    pltpu.make_async_copy(slab_ref, o_ref.at[0, 0], sem).start()
    pltpu.make_async_copy(slab_ref, o_ref.at[0, 0], sem).wait()


def kernel(x, table):
    B, T, N, _ = x.shape
    D = table.shape[1]
    t2 = table.T
    slab = _make_sc_lookup(D, N)(t2)
    out = pl.pallas_call(
        _make_tc_body(B, T),
        in_specs=[pl.BlockSpec(memory_space=pltpu.VMEM)],
        out_specs=pl.BlockSpec(memory_space=pl.ANY),
        out_shape=jax.ShapeDtypeStruct((B, T, D, N), jnp.float32),
        scratch_shapes=[pltpu.VMEM((2, D, N), jnp.float32), pltpu.SemaphoreType.DMA],
    )(t2)
    out = pl.pallas_call(
        _merge_body,
        in_specs=[
            pl.BlockSpec(memory_space=pl.ANY),
            pl.BlockSpec(memory_space=pltpu.VMEM),
        ],
        out_specs=pl.BlockSpec(memory_space=pl.ANY),
        out_shape=jax.ShapeDtypeStruct((B, T, D, N), jnp.float32),
        scratch_shapes=[pltpu.SemaphoreType.DMA],
        input_output_aliases={0: 0},
    )(out, slab)
    return out.transpose(0, 1, 3, 2)


# speedup vs baseline: 1.4017x; 1.0114x over previous
import functools

import jax
import jax.numpy as jnp
from jax import lax
from jax.experimental import pallas as pl
from jax.experimental.pallas import tpu as pltpu
from jax.experimental.pallas import tpu_sc as plsc

_NC, _NS = 2, 16


def _make_sc_lookup(D, N):
    n_chunks = D // 8
    mesh = plsc.VectorSubcoreMesh(
        core_axis_name="c", subcore_axis_name="s", num_cores=_NC, num_subcores=_NS
    )

    @functools.partial(
        pl.kernel,
        mesh=mesh,
        out_type=jax.ShapeDtypeStruct((D, N), jnp.float32),
        scratch_types=[
            pltpu.VMEM((8, N), jnp.float32),
            pltpu.SemaphoreType.DMA,
        ],
        compiler_params=pltpu.CompilerParams(use_tc_tiling_on_sc=True),
    )
    def k(t_hbm, o_hbm, buf, sem):
        c = lax.axis_index("c")
        s = lax.axis_index("s")
        wid = s * _NC + c

        @pl.when(wid < n_chunks)
        def _lookup():
            r0 = pl.multiple_of(wid * 8, 8)
            pltpu.make_async_copy(t_hbm.at[pl.ds(r0, 8)], buf, sem).start()
            pltpu.make_async_copy(t_hbm.at[pl.ds(r0, 8)], buf, sem).wait()
            pltpu.make_async_copy(buf, o_hbm.at[pl.ds(r0, 8)], sem).start()
            pltpu.make_async_copy(buf, o_hbm.at[pl.ds(r0, 8)], sem).wait()

    return k


def _make_tc_body(B, T):
    def body(t_ref, o_ref, sem):
        def fire(i, c):
            b = i // T
            t = i - b * T
            pltpu.make_async_copy(t_ref, o_ref.at[b, t], sem).start()
            return c

        lax.fori_loop(0, B * T, fire, 0)

        def drain(i, c):
            b = i // T
            t = i - b * T
            pltpu.make_async_copy(t_ref, o_ref.at[b, t], sem).wait()
            return c

        lax.fori_loop(0, B * T, drain, 0)

    return body


def _merge_body(big_ref, slab_ref, o_ref, sem):
    del big_ref
    pltpu.make_async_copy(slab_ref, o_ref.at[0, 0], sem).start()
    pltpu.make_async_copy(slab_ref, o_ref.at[0, 0], sem).wait()


def kernel(x, table):
    B, T, N, _ = x.shape
    D = table.shape[1]
    t2 = table.T
    slab = _make_sc_lookup(D, N)(t2)
    out = pl.pallas_call(
        _make_tc_body(B, T),
        in_specs=[pl.BlockSpec(memory_space=pltpu.VMEM)],
        out_specs=pl.BlockSpec(memory_space=pl.ANY),
        out_shape=jax.ShapeDtypeStruct((B, T, D, N), jnp.float32),
        scratch_shapes=[pltpu.SemaphoreType.DMA],
    )(t2)
    out = pl.pallas_call(
        _merge_body,
        in_specs=[
            pl.BlockSpec(memory_space=pl.ANY),
            pl.BlockSpec(memory_space=pltpu.VMEM),
        ],
        out_specs=pl.BlockSpec(memory_space=pl.ANY),
        out_shape=jax.ShapeDtypeStruct((B, T, D, N), jnp.float32),
        scratch_shapes=[pltpu.SemaphoreType.DMA],
        input_output_aliases={0: 0},
    )(out, slab)
    return out.transpose(0, 1, 3, 2)
